# baseline (speedup 1.0000x reference)
import jax
import jax.numpy as jnp
from jax.experimental import pallas as pl
from jax.experimental.pallas import tpu as pltpu


def _se_kernel(x_ref, w1t_ref, b1_ref, w2t_ref, b2_ref, o_ref, acc_ref):
    s = pl.program_id(1)

    @pl.when(s == 0)
    def _():
        acc_ref[...] = jnp.zeros_like(acc_ref)

    acc_ref[...] += jnp.sum(x_ref[...], axis=2)

    @pl.when(s == pl.num_programs(1) - 1)
    def _():
        pooled = acc_ref[...]
        h = jnp.dot(pooled, w1t_ref[...], preferred_element_type=jnp.float32)
        h = jnp.maximum(h + b1_ref[...], 0.0)
        y = jnp.dot(h, w2t_ref[...], preferred_element_type=jnp.float32)
        o_ref[...] = jax.nn.sigmoid(y + b2_ref[...])


def kernel(x, w1, b1, w2, b2):
    N, C, H, W = x.shape
    HW = H * W
    hid = w1.shape[0]

    x_flat = x.astype(jnp.float32).reshape(N, C, HW)

    w1t = w1.astype(jnp.float32).T / jnp.float32(HW)
    w2t = w2.astype(jnp.float32).T
    b1_2d = b1.astype(jnp.float32).reshape(1, hid)
    b2_2d = b2.astype(jnp.float32).reshape(1, C)

    max_elems = 4 * 1024 * 1024
    TN = min(8, N)
    while TN > 1 and TN * C * HW > max_elems:
        TN //= 2
    n_pad = -(-N // TN) * TN
    hw_tile = HW
    while TN * C * hw_tile > max_elems and hw_tile % 2 == 0:
        hw_tile //= 2
    hw_pad = -(-HW // hw_tile) * hw_tile

    if n_pad != N or hw_pad != HW:
        x_flat = jnp.pad(x_flat, ((0, n_pad - N), (0, 0), (0, hw_pad - HW)))

    grid = (n_pad // TN, hw_pad // hw_tile)

    out = pl.pallas_call(
        _se_kernel,
        out_shape=jax.ShapeDtypeStruct((n_pad, C), jnp.float32),
        grid=grid,
        in_specs=[
            pl.BlockSpec((TN, C, hw_tile), lambda n, s: (n, 0, s)),
            pl.BlockSpec((C, hid), lambda n, s: (0, 0)),
            pl.BlockSpec((1, hid), lambda n, s: (0, 0)),
            pl.BlockSpec((hid, C), lambda n, s: (0, 0)),
            pl.BlockSpec((1, C), lambda n, s: (0, 0)),
        ],
        out_specs=pl.BlockSpec((TN, C), lambda n, s: (n, 0)),
        scratch_shapes=[pltpu.VMEM((TN, C), jnp.float32)],
        compiler_params=pltpu.CompilerParams(
            dimension_semantics=("parallel", "arbitrary"),
            vmem_limit_bytes=48 * 1024 * 1024,
        ),
    )(x_flat, w1t, b1_2d, w2t, b2_2d)

    return out[:N].reshape(N, C, 1, 1)


# speedup vs baseline: 3.5938x; 3.5938x over previous
import jax
import jax.numpy as jnp
from jax.experimental import pallas as pl
from jax.experimental.pallas import tpu as pltpu


def _se_kernel(x_ref, w1t_ref, b1_ref, w2t_ref, b2_ref, o_ref, acc_ref):
    s = pl.program_id(1)
    ns = pl.num_programs(1)

    @pl.when(s == 0)
    def _():
        acc_ref[...] = jnp.zeros_like(acc_ref)

    acc_ref[...] += jnp.sum(x_ref[...], axis=1)

    @pl.when(s == ns - 1)
    def _():
        pooled = acc_ref[...]
        h = jnp.dot(pooled, w1t_ref[...], preferred_element_type=jnp.float32)
        h = jnp.maximum(h + b1_ref[...], 0.0)
        y = jnp.dot(h, w2t_ref[...], preferred_element_type=jnp.float32)
        o_ref[...] = jax.nn.sigmoid(y + b2_ref[...])


def kernel(x, w1, b1, w2, b2):
    N, C, H, W = x.shape
    HW = H * W
    hid = w1.shape[0]

    x_flat = jnp.transpose(x.astype(jnp.float32), (0, 2, 3, 1)).reshape(N, HW, C)

    w1t = w1.astype(jnp.float32).T / jnp.float32(HW)
    w2t = w2.astype(jnp.float32).T
    b1_2d = b1.astype(jnp.float32).reshape(1, hid)
    b2_2d = b2.astype(jnp.float32).reshape(1, C)

    max_elems = 4 * 1024 * 1024
    TN = min(8, N)
    while TN > 1 and TN * C * HW > max_elems:
        TN //= 2
    n_pad = -(-N // TN) * TN
    hw_tile = HW
    while TN * hw_tile * C > max_elems and hw_tile % 2 == 0:
        hw_tile //= 2
    hw_pad = -(-HW // hw_tile) * hw_tile

    if n_pad != N or hw_pad != HW:
        x_flat = jnp.pad(x_flat, ((0, n_pad - N), (0, hw_pad - HW), (0, 0)))

    grid = (n_pad // TN, hw_pad // hw_tile)

    out = pl.pallas_call(
        _se_kernel,
        out_shape=jax.ShapeDtypeStruct((n_pad, C), jnp.float32),
        grid=grid,
        in_specs=[
            pl.BlockSpec((TN, hw_tile, C), lambda n, s: (n, s, 0)),
            pl.BlockSpec((C, hid), lambda n, s: (0, 0)),
            pl.BlockSpec((1, hid), lambda n, s: (0, 0)),
            pl.BlockSpec((hid, C), lambda n, s: (0, 0)),
            pl.BlockSpec((1, C), lambda n, s: (0, 0)),
        ],
        out_specs=pl.BlockSpec((TN, C), lambda n, s: (n, 0)),
        scratch_shapes=[pltpu.VMEM((TN, C), jnp.float32)],
        compiler_params=pltpu.CompilerParams(
            dimension_semantics=("parallel", "arbitrary"),
            vmem_limit_bytes=64 * 1024 * 1024,
        ),
    )(x_flat, w1t, b1_2d, w2t, b2_2d)

    return out[:N].reshape(N, C, 1, 1)
